# baseline (speedup 1.0000x reference)
import functools

import jax
import jax.numpy as jnp
from jax import lax
from jax.experimental import pallas as pl
from jax.experimental.pallas import tpu as pltpu
from jax.experimental.pallas import tpu_sc as plsc

EMBED_DIM = 32
NUM_CORES = 2
NUM_SUBCORES = 16
NUM_WORKERS = NUM_CORES * NUM_SUBCORES
IDX_PAD = 32
RPS = 32


def _tc_unflatten(flat, batch, n_fields):
    rows_per_step = 64
    grid = batch // rows_per_step

    def body(x_ref, o_ref):
        for r in range(rows_per_step):
            y = x_ref[pl.ds(r * IDX_PAD, IDX_PAD), :]
            o_ref[r] = y[:n_fields, :]

    return pl.pallas_call(
        body,
        out_shape=jax.ShapeDtypeStruct((batch, n_fields, EMBED_DIM), flat.dtype),
        grid=(grid,),
        in_specs=[pl.BlockSpec((rows_per_step * IDX_PAD, EMBED_DIM),
                               lambda i: (i, 0))],
        out_specs=pl.BlockSpec((rows_per_step, n_fields, EMBED_DIM),
                               lambda i: (i, 0, 0)),
        compiler_params=pltpu.CompilerParams(
            dimension_semantics=("parallel",)),
    )(flat)


def _sc_gather(table, idx_padded, n_fields):
    batch = idx_padded.shape[0]
    num_flat = batch * IDX_PAD
    rows_per_worker = batch // NUM_WORKERS
    n_super = rows_per_worker // RPS
    assert n_super % 2 == 0
    mesh = plsc.VectorSubcoreMesh(core_axis_name="c", subcore_axis_name="s")

    @functools.partial(
        pl.kernel,
        out_type=jax.ShapeDtypeStruct((num_flat, EMBED_DIM), table.dtype),
        mesh=mesh,
        scratch_types=[
            pltpu.VMEM((RPS, IDX_PAD), jnp.int32),
            pltpu.VMEM((RPS, IDX_PAD), jnp.int32),
            pltpu.VMEM((RPS * IDX_PAD, EMBED_DIM), jnp.float32),
            pltpu.VMEM((RPS * IDX_PAD, EMBED_DIM), jnp.float32),
            pltpu.SemaphoreType.DMA,
            pltpu.SemaphoreType.DMA,
            pltpu.SemaphoreType.DMA,
            pltpu.SemaphoreType.DMA,
            pltpu.SemaphoreType.DMA,
            pltpu.SemaphoreType.DMA,
        ],
        compiler_params=pltpu.CompilerParams(use_tc_tiling_on_sc=False),
    )
    def gather_kernel(table_hbm, idx_hbm, out_hbm, i0, i1, r0, r1,
                      isem0, isem1, gsem0, gsem1, osem0, osem1):
        wid = lax.axis_index("s") * NUM_CORES + lax.axis_index("c")
        base = wid * rows_per_worker
        fbase = base * IDX_PAD
        fsuper = RPS * IDX_PAD
        idx_bufs = (i0, i1)
        row_bufs = (r0, r1)
        isems = (isem0, isem1)
        gsems = (gsem0, gsem1)
        osems = (osem0, osem1)

        pltpu.async_copy(idx_hbm.at[pl.ds(base, RPS)], i0, isem0)
        pltpu.async_copy(idx_hbm.at[pl.ds(base + RPS, RPS)], i1, isem1)

        @pl.loop(0, n_super // 2)
        def _(u):
            for b in range(2):
                ib, rb = idx_bufs[b], row_bufs[b]
                s = 2 * u + b

                @pl.when(u >= 1)
                def _():
                    pltpu.make_async_copy(
                        out_hbm.at[pl.ds(fbase, fsuper)], rb,
                        osems[b]).wait()

                pltpu.make_async_copy(
                    idx_hbm.at[pl.ds(base, RPS)], ib, isems[b]).wait()

                for j in range(RPS):
                    pltpu.async_copy(
                        table_hbm.at[ib.at[j]],
                        rb.at[pl.ds(j * IDX_PAD, IDX_PAD)], gsems[b])

                pltpu.make_async_copy(
                    out_hbm.at[pl.ds(fbase, fsuper)], rb, gsems[b]).wait()
                pltpu.async_copy(
                    rb, out_hbm.at[pl.ds(fbase + s * fsuper, fsuper)], osems[b])

                @pl.when(u < n_super // 2 - 1)
                def _():
                    pltpu.async_copy(
                        idx_hbm.at[pl.ds(base + (s + 2) * RPS, RPS)],
                        ib, isems[b])

        pltpu.make_async_copy(
            out_hbm.at[pl.ds(fbase, fsuper)], r0, osem0).wait()
        pltpu.make_async_copy(
            out_hbm.at[pl.ds(fbase, fsuper)], r1, osem1).wait()

    return gather_kernel(table, idx_padded)


@jax.jit
def kernel(inputs, indices):
    batch, n_fields = indices.shape
    idx_padded = jnp.pad(indices, ((0, 0), (0, IDX_PAD - n_fields)))
    flat = _sc_gather(inputs, idx_padded, n_fields)
    return _tc_unflatten(flat, batch, n_fields)


# speedup vs baseline: 1.8809x; 1.8809x over previous
import functools

import jax
import jax.numpy as jnp
from jax import lax
from jax.experimental import pallas as pl
from jax.experimental.pallas import tpu as pltpu
from jax.experimental.pallas import tpu_sc as plsc

EMBED_DIM = 32
NUM_CORES = 2
NUM_SUBCORES = 16
NUM_WORKERS = NUM_CORES * NUM_SUBCORES
IDX_PAD = 32
RPS = 32


def _tc_unflatten(flat, batch, n_fields):
    rows_per_step = 64
    grid = batch // rows_per_step

    def body(x_ref, o_ref):
        for r in range(rows_per_step):
            y = x_ref[pl.ds(r * IDX_PAD, IDX_PAD), :]
            o_ref[r] = y[:n_fields, :]

    return pl.pallas_call(
        body,
        out_shape=jax.ShapeDtypeStruct((batch, n_fields, EMBED_DIM), flat.dtype),
        grid=(grid,),
        in_specs=[pl.BlockSpec((rows_per_step * IDX_PAD, EMBED_DIM),
                               lambda i: (i, 0))],
        out_specs=pl.BlockSpec((rows_per_step, n_fields, EMBED_DIM),
                               lambda i: (i, 0, 0)),
        compiler_params=pltpu.CompilerParams(
            dimension_semantics=("parallel",)),
    )(flat)


def _sc_gather(table, idx_padded, n_fields):
    batch = idx_padded.shape[0]
    num_flat = batch * IDX_PAD
    rows_per_worker = batch // NUM_WORKERS
    n_super = rows_per_worker // RPS
    assert n_super % 2 == 0
    mesh = plsc.VectorSubcoreMesh(core_axis_name="c", subcore_axis_name="s")

    @functools.partial(
        pl.kernel,
        out_type=jax.ShapeDtypeStruct((num_flat, EMBED_DIM), table.dtype),
        mesh=mesh,
        scratch_types=[
            pltpu.VMEM((RPS, IDX_PAD), jnp.int32),
            pltpu.VMEM((RPS, IDX_PAD), jnp.int32),
            pltpu.VMEM((RPS * IDX_PAD, EMBED_DIM), jnp.float32),
            pltpu.VMEM((RPS * IDX_PAD, EMBED_DIM), jnp.float32),
            pltpu.SemaphoreType.DMA,
            pltpu.SemaphoreType.DMA,
            pltpu.SemaphoreType.DMA,
            pltpu.SemaphoreType.DMA,
            pltpu.SemaphoreType.DMA,
            pltpu.SemaphoreType.DMA,
        ],
        compiler_params=pltpu.CompilerParams(use_tc_tiling_on_sc=False),
    )
    def gather_kernel(table_hbm, idx_hbm, out_hbm, i0, i1, r0, r1,
                      isem0, isem1, gsem0, gsem1, osem0, osem1):
        wid = lax.axis_index("s") * NUM_CORES + lax.axis_index("c")
        base = wid * rows_per_worker
        fbase = base * IDX_PAD
        fsuper = RPS * IDX_PAD
        idx_bufs = (i0, i1)
        row_bufs = (r0, r1)
        isems = (isem0, isem1)
        gsems = (gsem0, gsem1)
        osems = (osem0, osem1)

        pltpu.async_copy(idx_hbm.at[pl.ds(base, RPS)], i0, isem0)
        pltpu.async_copy(idx_hbm.at[pl.ds(base + RPS, RPS)], i1, isem1)

        @pl.loop(0, n_super // 2)
        def _(u):
            for b in range(2):
                ib, rb = idx_bufs[b], row_bufs[b]
                s = 2 * u + b

                @pl.when(u >= 1)
                def _():
                    pltpu.make_async_copy(
                        out_hbm.at[pl.ds(fbase, fsuper)], rb,
                        osems[b]).wait()

                pltpu.make_async_copy(
                    idx_hbm.at[pl.ds(base, RPS)], ib, isems[b]).wait()

                for j in range(RPS):
                    pltpu.async_copy(
                        table_hbm.at[ib.at[j]],
                        rb.at[pl.ds(j * IDX_PAD, IDX_PAD)], gsems[b])

                pltpu.make_async_copy(
                    out_hbm.at[pl.ds(fbase, fsuper)], rb, gsems[b]).wait()
                pltpu.async_copy(
                    rb, out_hbm.at[pl.ds(fbase + s * fsuper, fsuper)], osems[b])

                @pl.when(u < n_super // 2 - 1)
                def _():
                    pltpu.async_copy(
                        idx_hbm.at[pl.ds(base + (s + 2) * RPS, RPS)],
                        ib, isems[b])

        pltpu.make_async_copy(
            out_hbm.at[pl.ds(fbase, fsuper)], r0, osem0).wait()
        pltpu.make_async_copy(
            out_hbm.at[pl.ds(fbase, fsuper)], r1, osem1).wait()

    return gather_kernel(table, idx_padded)


@jax.jit
def kernel(inputs, indices):
    batch, n_fields = indices.shape
    junk = jnp.arange(batch * (IDX_PAD - n_fields), dtype=indices.dtype)
    junk = junk.reshape(batch, IDX_PAD - n_fields) % inputs.shape[0]
    idx_padded = jnp.concatenate([indices, junk], axis=1)
    flat = _sc_gather(inputs, idx_padded, n_fields)
    return _tc_unflatten(flat, batch, n_fields)


# speedup vs baseline: 2.7772x; 1.4765x over previous
import functools

import jax
import jax.numpy as jnp
from jax import lax
from jax.experimental import pallas as pl
from jax.experimental.pallas import tpu as pltpu
from jax.experimental.pallas import tpu_sc as plsc

EMBED_DIM = 32
NUM_CORES = 2
NUM_SUBCORES = 16
NUM_WORKERS = NUM_CORES * NUM_SUBCORES
RPS = 32


def _sc_gather(table, indices):
    batch, n_fields = indices.shape
    rows_per_worker = batch // NUM_WORKERS
    n_super = rows_per_worker // RPS
    assert n_super % 2 == 0
    mesh = plsc.VectorSubcoreMesh(core_axis_name="c", subcore_axis_name="s")

    @functools.partial(
        pl.kernel,
        out_type=jax.ShapeDtypeStruct((batch, n_fields, EMBED_DIM), table.dtype),
        mesh=mesh,
        scratch_types=[
            pltpu.VMEM((RPS, n_fields), jnp.int32),
            pltpu.VMEM((RPS, n_fields), jnp.int32),
            pltpu.VMEM((RPS, n_fields, EMBED_DIM), jnp.float32),
            pltpu.VMEM((RPS, n_fields, EMBED_DIM), jnp.float32),
            pltpu.SemaphoreType.DMA,
            pltpu.SemaphoreType.DMA,
            pltpu.SemaphoreType.DMA,
            pltpu.SemaphoreType.DMA,
            pltpu.SemaphoreType.DMA,
            pltpu.SemaphoreType.DMA,
        ],
        compiler_params=pltpu.CompilerParams(use_tc_tiling_on_sc=False),
    )
    def gather_kernel(table_hbm, idx_hbm, out_hbm, i0, i1, r0, r1,
                      isem0, isem1, gsem0, gsem1, osem0, osem1):
        wid = lax.axis_index("s") * NUM_CORES + lax.axis_index("c")
        base = wid * rows_per_worker
        idx_bufs = (i0, i1)
        row_bufs = (r0, r1)
        isems = (isem0, isem1)
        gsems = (gsem0, gsem1)
        osems = (osem0, osem1)

        pltpu.async_copy(idx_hbm.at[pl.ds(base, RPS)], i0, isem0)
        pltpu.async_copy(idx_hbm.at[pl.ds(base + RPS, RPS)], i1, isem1)

        @pl.loop(0, n_super // 2)
        def _(u):
            for b in range(2):
                ib, rb = idx_bufs[b], row_bufs[b]
                s = 2 * u + b
                off = base + s * RPS

                @pl.when(u >= 1)
                def _():
                    pltpu.make_async_copy(
                        out_hbm.at[pl.ds(base, RPS)], rb, osems[b]).wait()

                pltpu.make_async_copy(
                    idx_hbm.at[pl.ds(base, RPS)], ib, isems[b]).wait()

                for j in range(RPS):
                    pltpu.async_copy(
                        table_hbm.at[ib.at[j]], rb.at[j], gsems[b])

                pltpu.make_async_copy(
                    out_hbm.at[pl.ds(base, RPS)], rb, gsems[b]).wait()
                pltpu.async_copy(rb, out_hbm.at[pl.ds(off, RPS)], osems[b])

                @pl.when(u < n_super // 2 - 1)
                def _():
                    pltpu.async_copy(
                        idx_hbm.at[pl.ds(base + (s + 2) * RPS, RPS)],
                        ib, isems[b])

        pltpu.make_async_copy(
            out_hbm.at[pl.ds(base, RPS)], r0, osem0).wait()
        pltpu.make_async_copy(
            out_hbm.at[pl.ds(base, RPS)], r1, osem1).wait()

    return gather_kernel(table, indices)


@jax.jit
def kernel(inputs, indices):
    return _sc_gather(inputs, indices)
